# Initial kernel scaffold; baseline (speedup 1.0000x reference)
import jax
import jax.numpy as jnp
from jax.experimental import pallas as pl


def kernel(x, edge_index, gat_fc_W, gat_fc_b, attn_l, attn_r, sem_W1, sem_b1, sem_W2, gate_W, gate_b, l2W, l2b, l3W, l3b, l4W, l4b, yce_W, yce_b, bn1_g, bn1_b, bn2_g, bn2_b, bn3_g, bn3_b, fus_Wp, fus_b, att_W1, att_b1, att_w2):
    raise NotImplementedError("write your pallas kernel here")


# speedup vs baseline: 1.0001x; 1.0001x over previous
import jax
import jax.numpy as jnp
from jax.experimental import pallas as pl

N = 10000
E = 160000
IN = 768
HEADS = 8
HID = 64
D = HEADS * HID


def _bn(x, g, b):
    m = x.mean(0)
    v = x.var(0)
    return (x - m) / jnp.sqrt(v + 1e-5) * g + b


def _ln(x):
    m = x.mean(-1, keepdims=True)
    v = x.var(-1, keepdims=True)
    return (x - m) / jnp.sqrt(v + 1e-5)


def _edge_softmax(e, dst, n):
    emax = jax.ops.segment_max(e, dst, num_segments=n)
    emax = jnp.where(jnp.isfinite(emax), emax, 0.0)
    ex = jnp.exp(e - emax[dst])
    s = jax.ops.segment_sum(ex, dst, num_segments=n)
    return ex / (s[dst] + 1e-9)


def _mm_kernel(a_ref, b_ref, o_ref):
    o_ref[...] = jnp.dot(a_ref[...], b_ref[...], preferred_element_type=jnp.float32)


def _pallas_mm(a, b, bm=2000):
    m, k = a.shape
    n = b.shape[1]
    return pl.pallas_call(
        _mm_kernel,
        grid=(m // bm,),
        in_specs=[
            pl.BlockSpec((bm, k), lambda i: (i, 0)),
            pl.BlockSpec((k, n), lambda i: (0, 0)),
        ],
        out_specs=pl.BlockSpec((bm, n), lambda i: (i, 0)),
        out_shape=jax.ShapeDtypeStruct((m, n), jnp.float32),
    )(a, b)


def kernel(x, edge_index, gat_fc_W, gat_fc_b, attn_l, attn_r, sem_W1, sem_b1, sem_W2, gate_W, gate_b, l2W, l2b, l3W, l3b, l4W, l4b, yce_W, yce_b, bn1_g, bn1_b, bn2_g, bn2_b, bn3_g, bn3_b, fus_Wp, fus_b, att_W1, att_b1, att_w2):
    src = edge_index[0]
    dst = edge_index[1]
    feat = (x @ gat_fc_W).reshape(N, HEADS, HID)
    el = (feat * attn_l[None]).sum(-1)
    er = (feat * attn_r[None]).sum(-1)
    e = jax.nn.leaky_relu(el[src] + er[dst], 0.2)
    alpha = _edge_softmax(e, dst, N)
    msg = feat[src] * alpha[:, :, None]
    rst = jax.ops.segment_sum(msg, dst, num_segments=N) + gat_fc_b.reshape(HEADS, HID)[None]
    gat_out = jax.nn.elu(rst).reshape(N, D)
    h1 = gat_out
    h3 = x @ l3W + l3b
    hh = jax.nn.elu(_bn(h3 + h1, bn1_g, bn1_b))
    gate = hh @ gate_W + gate_b
    a = jax.nn.softmax(gate, axis=0)
    Tp = (a * hh).sum(0, keepdims=True)
    T = Tp @ l4W + l4b
    proj = hh @ fus_Wp + fus_b
    score = (_ln(x) * _ln(proj)).sum(-1, keepdims=True) / jnp.sqrt(768.0)
    gamma = jax.nn.sigmoid(score)
    newT = _bn(gamma * x + (1.0 - gamma) * proj, bn3_g, bn3_b)
    stack = jnp.stack([jnp.broadcast_to(T, newT.shape), newT], axis=1)
    s = jnp.tanh(stack @ att_W1 + att_b1) @ att_w2
    al = jax.nn.softmax(s, axis=1)
    RH = (al[..., None] * stack).sum(1)
    Tel = jax.nn.elu(_bn(_pallas_mm(RH, l2W) + l2b, bn2_g, bn2_b))
    return Tel @ yce_W + yce_b


# speedup vs baseline: 9.9902x; 9.9895x over previous
import functools

import jax
import jax.numpy as jnp
from jax import lax
from jax.experimental import pallas as pl
from jax.experimental.pallas import tpu as pltpu
from jax.experimental.pallas import tpu_sc as plsc

N = 10000
E = 160000
IN = 768
HEADS = 8
HID = 64
D = HEADS * HID
WA = 640
WB = 128
NPL = 5

_NC = 2
_NS = 16
_NW = _NC * _NS
E2 = 163840
EW = E2 // _NW
TF = 64
NTF = EW // TF
TE = 256
NTE = EW // TE
TS = 256
NTS = EW // TS
NP = 10240
STRIPE = NP // _NS

_mesh = plsc.VectorSubcoreMesh(core_axis_name="c", subcore_axis_name="s")


@functools.partial(
    pl.kernel,
    mesh=_mesh,
    out_type=[
        jax.ShapeDtypeStruct((E2, WA), jnp.float32),
        jax.ShapeDtypeStruct((E2, WB), jnp.float32),
    ],
    scratch_types=[
        pltpu.VMEM((EW,), jnp.int32),
        pltpu.VMEM((EW,), jnp.int32),
        pltpu.VMEM((TF, WA), jnp.float32),
        pltpu.VMEM((TE, WB), jnp.float32),
        pltpu.SemaphoreType.DMA,
    ],
)
def _sc_gather(ta_hbm, tb_hbm, srcp_hbm, dstp_hbm,
               fa_out, erd_out, src_v, dst_v, rows_v, small_v, sem):
    c = lax.axis_index("c")
    s = lax.axis_index("s")
    wid = s * _NC + c
    base = wid * EW
    pltpu.sync_copy(srcp_hbm.at[pl.ds(base, EW)], src_v)
    pltpu.sync_copy(dstp_hbm.at[pl.ds(base, EW)], dst_v)
    for t in range(NTF):
        pltpu.async_copy(ta_hbm.at[src_v.at[pl.ds(t * TF, TF)]], rows_v, sem).wait()
        pltpu.sync_copy(rows_v, fa_out.at[pl.ds(base + t * TF, TF)])
    for t in range(NTE):
        pltpu.async_copy(tb_hbm.at[dst_v.at[pl.ds(t * TE, TE)]], small_v, sem).wait()
        pltpu.sync_copy(small_v, erd_out.at[pl.ds(base + t * TE, TE)])


@functools.partial(
    pl.kernel,
    mesh=_mesh,
    out_type=jax.ShapeDtypeStruct((_NC, NPL, NP, WB), jnp.float32),
    scratch_types=[
        pltpu.VMEM((TS,), jnp.int32),
        pltpu.VMEM((TS, WB), jnp.float32),
        pltpu.VMEM_SHARED((NP, WB), jnp.float32),
    ],
)
def _sc_scatter(mx_hbm, dst3_hbm, zacc_hbm, acc_out, dst_v, m_v, acc_sh):
    c = lax.axis_index("c")
    s = lax.axis_index("s")
    wid = s * _NC + c
    base = wid * EW
    for p in range(NPL):
        pltpu.sync_copy(zacc_hbm, acc_sh.at[pl.ds(s * STRIPE, STRIPE)])
        plsc.subcore_barrier()
        for t in range(NTS):
            pltpu.sync_copy(dst3_hbm.at[wid].at[t], dst_v)
            pltpu.sync_copy(mx_hbm.at[p].at[pl.ds(base + t * TS, TS)], m_v)
            pltpu.sync_copy(m_v, acc_sh.at[dst_v], add=True)
        plsc.subcore_barrier()
        pltpu.sync_copy(acc_sh.at[pl.ds(s * STRIPE, STRIPE)],
                        acc_out.at[c].at[p].at[pl.ds(s * STRIPE, STRIPE)])
        plsc.subcore_barrier()


def _feat_kernel(x_ref, w_ref, al_ref, ar_ref, ta_ref, tb_ref):
    bm = x_ref.shape[0]
    f = jnp.dot(x_ref[...], w_ref[...], preferred_element_type=jnp.float32)
    f3 = f.reshape(bm, HEADS, HID)
    el = (f3 * al_ref[...][None]).sum(-1)
    er = (f3 * ar_ref[...][None]).sum(-1)
    zpad = jnp.zeros((bm, WB - HEADS), jnp.float32)
    ta_ref[...] = jnp.concatenate([f, el, zpad[:, :WA - D - HEADS]], axis=1)
    tb_ref[...] = jnp.concatenate([er, zpad], axis=1)


def _feat_tables(x, w, al, ar, bm=2000):
    return pl.pallas_call(
        _feat_kernel,
        grid=(N // bm,),
        in_specs=[
            pl.BlockSpec((bm, IN), lambda i: (i, 0)),
            pl.BlockSpec((IN, D), lambda i: (0, 0)),
            pl.BlockSpec((HEADS, HID), lambda i: (0, 0)),
            pl.BlockSpec((HEADS, HID), lambda i: (0, 0)),
        ],
        out_specs=[
            pl.BlockSpec((bm, WA), lambda i: (i, 0)),
            pl.BlockSpec((bm, WB), lambda i: (i, 0)),
        ],
        out_shape=[
            jax.ShapeDtypeStruct((N, WA), jnp.float32),
            jax.ShapeDtypeStruct((N, WB), jnp.float32),
        ],
    )(x, w, al, ar)


def _edge_mul_kernel(fa_ref, erd_ref, mx_ref):
    bm = fa_ref.shape[0]
    fa = fa_ref[...]
    e = fa[:, D:D + HEADS] + erd_ref[...][:, :HEADS]
    e = jnp.where(e > 0, e, 0.2 * e)
    ex = jnp.exp(e)
    for p in range(NPL - 1):
        h0, h1 = 2 * p, 2 * p + 1
        mx_ref[p] = jnp.concatenate(
            [fa[:, h0 * HID:(h0 + 1) * HID] * ex[:, h0:h0 + 1],
             fa[:, h1 * HID:(h1 + 1) * HID] * ex[:, h1:h1 + 1]], axis=1)
    mx_ref[NPL - 1] = jnp.concatenate(
        [ex, jnp.zeros((bm, WB - HEADS), jnp.float32)], axis=1)


def _edge_mul(fa, erd, bm=2048):
    return pl.pallas_call(
        _edge_mul_kernel,
        grid=(E2 // bm,),
        in_specs=[
            pl.BlockSpec((bm, WA), lambda i: (i, 0)),
            pl.BlockSpec((bm, WB), lambda i: (i, 0)),
        ],
        out_specs=pl.BlockSpec((NPL, bm, WB), lambda i: (0, i, 0)),
        out_shape=jax.ShapeDtypeStruct((NPL, E2, WB), jnp.float32),
    )(fa, erd)


def _mm_kernel(a_ref, b_ref, o_ref):
    o_ref[...] = jnp.dot(a_ref[...], b_ref[...], preferred_element_type=jnp.float32)


def _pallas_mm(a, b, bm=2000):
    m, k = a.shape
    n = b.shape[1]
    return pl.pallas_call(
        _mm_kernel,
        grid=(m // bm,),
        in_specs=[
            pl.BlockSpec((bm, k), lambda i: (i, 0)),
            pl.BlockSpec((k, n), lambda i: (0, 0)),
        ],
        out_specs=pl.BlockSpec((bm, n), lambda i: (i, 0)),
        out_shape=jax.ShapeDtypeStruct((m, n), jnp.float32),
    )(a, b)


def _bn(x, g, b):
    m = x.mean(0)
    v = x.var(0)
    return (x - m) / jnp.sqrt(v + 1e-5) * g + b


def _ln(x):
    m = x.mean(-1, keepdims=True)
    v = x.var(-1, keepdims=True)
    return (x - m) / jnp.sqrt(v + 1e-5)


def kernel(x, edge_index, gat_fc_W, gat_fc_b, attn_l, attn_r, sem_W1, sem_b1, sem_W2, gate_W, gate_b, l2W, l2b, l3W, l3b, l4W, l4b, yce_W, yce_b, bn1_g, bn1_b, bn2_g, bn2_b, bn3_g, bn3_b, fus_Wp, fus_b, att_W1, att_b1, att_w2):
    src = edge_index[0]
    dst = edge_index[1]
    ta, tb = _feat_tables(x, gat_fc_W, attn_l, attn_r)
    tb_p = jnp.pad(tb, ((0, NP - N), (0, 0)))
    pad = E2 - E
    src_p = jnp.concatenate([src, jnp.zeros((pad,), jnp.int32)])
    dst_p = jnp.concatenate([dst, jnp.full((pad,), N, jnp.int32)])
    dst3 = dst_p.reshape(_NW, NTS, TS)
    fa, erd = _sc_gather(ta, tb_p, src_p, dst_p)
    mx = _edge_mul(fa, erd)
    zacc = jnp.zeros((STRIPE, WB), jnp.float32)
    parts = _sc_scatter(mx, dst3, zacc).sum(0)
    ssum = parts[NPL - 1][:N, :HEADS]
    agg = parts[:NPL - 1, :N, :]
    rst = agg.reshape(NPL - 1, N, 2, HID).transpose(1, 0, 2, 3).reshape(N, HEADS, HID)
    rst = rst / (ssum[:, :, None] + 1e-9)
    rst = rst + gat_fc_b.reshape(HEADS, HID)[None]
    gat_out = jax.nn.elu(rst).reshape(N, D)
    h3 = _pallas_mm(x, l3W) + l3b
    hh = jax.nn.elu(_bn(h3 + gat_out, bn1_g, bn1_b))
    gate = hh @ gate_W + gate_b
    a = jax.nn.softmax(gate, axis=0)
    Tp = (a * hh).sum(0, keepdims=True)
    T = Tp @ l4W + l4b
    proj = _pallas_mm(hh, fus_Wp) + fus_b
    score = (_ln(x) * _ln(proj)).sum(-1, keepdims=True) / jnp.sqrt(768.0)
    gamma = jax.nn.sigmoid(score)
    newT = _bn(gamma * x + (1.0 - gamma) * proj, bn3_g, bn3_b)
    s0 = jnp.tanh(T @ att_W1 + att_b1) @ att_w2
    s1 = jnp.tanh(_pallas_mm(newT, att_W1) + att_b1) @ att_w2
    sm = jnp.maximum(s0, s1)
    e0 = jnp.exp(s0 - sm)
    e1 = jnp.exp(s1 - sm)
    den = e0 + e1
    RH = (e0 / den)[:, None] * T + (e1 / den)[:, None] * newT
    Tel = jax.nn.elu(_bn(_pallas_mm(RH, l2W) + l2b, bn2_g, bn2_b))
    return Tel @ yce_W + yce_b


# speedup vs baseline: 9.9923x; 1.0002x over previous
import functools

import jax
import jax.numpy as jnp
from jax import lax
from jax.experimental import pallas as pl
from jax.experimental.pallas import tpu as pltpu
from jax.experimental.pallas import tpu_sc as plsc

N = 10000
E = 160000
IN = 768
HEADS = 8
HID = 64
D = HEADS * HID
WA = 640
WB = 128
NPL = 5

_NC = 2
_NS = 16
_NW = _NC * _NS
E2 = 163840
EW = E2 // _NW
TF = 128
NTF = EW // TF
TE = 256
NTE = EW // TE
TS = 256
NTS = EW // TS
NP = 10240
STRIPE = NP // _NS

_mesh = plsc.VectorSubcoreMesh(core_axis_name="c", subcore_axis_name="s")


@functools.partial(
    pl.kernel,
    mesh=_mesh,
    out_type=[
        jax.ShapeDtypeStruct((E2, WA), jnp.float32),
        jax.ShapeDtypeStruct((E2, WB), jnp.float32),
    ],
    scratch_types=[
        pltpu.VMEM((EW,), jnp.int32),
        pltpu.VMEM((EW,), jnp.int32),
        pltpu.VMEM((TF, WA), jnp.float32),
        pltpu.VMEM((TE, WB), jnp.float32),
        pltpu.SemaphoreType.DMA,
    ],
)
def _sc_gather(ta_hbm, tb_hbm, srcp_hbm, dstp_hbm,
               fa_out, erd_out, src_v, dst_v, rows_v, small_v, sem):
    c = lax.axis_index("c")
    s = lax.axis_index("s")
    wid = s * _NC + c
    base = wid * EW
    pltpu.sync_copy(srcp_hbm.at[pl.ds(base, EW)], src_v)
    pltpu.sync_copy(dstp_hbm.at[pl.ds(base, EW)], dst_v)
    for t in range(NTF):
        pltpu.async_copy(ta_hbm.at[src_v.at[pl.ds(t * TF, TF)]], rows_v, sem).wait()
        pltpu.sync_copy(rows_v, fa_out.at[pl.ds(base + t * TF, TF)])
    for t in range(NTE):
        pltpu.async_copy(tb_hbm.at[dst_v.at[pl.ds(t * TE, TE)]], small_v, sem).wait()
        pltpu.sync_copy(small_v, erd_out.at[pl.ds(base + t * TE, TE)])


@functools.partial(
    pl.kernel,
    mesh=_mesh,
    out_type=jax.ShapeDtypeStruct((_NC, NPL, NP, WB), jnp.float32),
    scratch_types=[
        pltpu.VMEM((TS,), jnp.int32),
        pltpu.VMEM((TS, WB), jnp.float32),
        pltpu.VMEM_SHARED((NP, WB), jnp.float32),
    ],
)
def _sc_scatter(mx_hbm, dst3_hbm, zacc_hbm, acc_out, dst_v, m_v, acc_sh):
    c = lax.axis_index("c")
    s = lax.axis_index("s")
    wid = s * _NC + c
    base = wid * EW
    for p in range(NPL):
        pltpu.sync_copy(zacc_hbm, acc_sh.at[pl.ds(s * STRIPE, STRIPE)])
        plsc.subcore_barrier()
        for t in range(NTS):
            pltpu.sync_copy(dst3_hbm.at[wid].at[t], dst_v)
            pltpu.sync_copy(mx_hbm.at[p].at[pl.ds(base + t * TS, TS)], m_v)
            pltpu.sync_copy(m_v, acc_sh.at[dst_v], add=True)
        plsc.subcore_barrier()
        pltpu.sync_copy(acc_sh.at[pl.ds(s * STRIPE, STRIPE)],
                        acc_out.at[c].at[p].at[pl.ds(s * STRIPE, STRIPE)])
        plsc.subcore_barrier()


def _feat_kernel(x_ref, w_ref, al_ref, ar_ref, ta_ref, tb_ref):
    bm = x_ref.shape[0]
    f = jnp.dot(x_ref[...], w_ref[...], preferred_element_type=jnp.float32)
    f3 = f.reshape(bm, HEADS, HID)
    el = (f3 * al_ref[...][None]).sum(-1)
    er = (f3 * ar_ref[...][None]).sum(-1)
    zpad = jnp.zeros((bm, WB - HEADS), jnp.float32)
    ta_ref[...] = jnp.concatenate([f, el, zpad[:, :WA - D - HEADS]], axis=1)
    tb_ref[...] = jnp.concatenate([er, zpad], axis=1)


def _feat_tables(x, w, al, ar, bm=2000):
    return pl.pallas_call(
        _feat_kernel,
        grid=(N // bm,),
        in_specs=[
            pl.BlockSpec((bm, IN), lambda i: (i, 0)),
            pl.BlockSpec((IN, D), lambda i: (0, 0)),
            pl.BlockSpec((HEADS, HID), lambda i: (0, 0)),
            pl.BlockSpec((HEADS, HID), lambda i: (0, 0)),
        ],
        out_specs=[
            pl.BlockSpec((bm, WA), lambda i: (i, 0)),
            pl.BlockSpec((bm, WB), lambda i: (i, 0)),
        ],
        out_shape=[
            jax.ShapeDtypeStruct((N, WA), jnp.float32),
            jax.ShapeDtypeStruct((N, WB), jnp.float32),
        ],
    )(x, w, al, ar)


def _edge_mul_kernel(fa_ref, erd_ref, mx_ref):
    bm = fa_ref.shape[0]
    fa = fa_ref[...]
    e = fa[:, D:D + HEADS] + erd_ref[...][:, :HEADS]
    e = jnp.where(e > 0, e, 0.2 * e)
    ex = jnp.exp(e)
    for p in range(NPL - 1):
        h0, h1 = 2 * p, 2 * p + 1
        mx_ref[p] = jnp.concatenate(
            [fa[:, h0 * HID:(h0 + 1) * HID] * ex[:, h0:h0 + 1],
             fa[:, h1 * HID:(h1 + 1) * HID] * ex[:, h1:h1 + 1]], axis=1)
    mx_ref[NPL - 1] = jnp.concatenate(
        [ex, jnp.zeros((bm, WB - HEADS), jnp.float32)], axis=1)


def _edge_mul(fa, erd, bm=2048):
    return pl.pallas_call(
        _edge_mul_kernel,
        grid=(E2 // bm,),
        in_specs=[
            pl.BlockSpec((bm, WA), lambda i: (i, 0)),
            pl.BlockSpec((bm, WB), lambda i: (i, 0)),
        ],
        out_specs=pl.BlockSpec((NPL, bm, WB), lambda i: (0, i, 0)),
        out_shape=jax.ShapeDtypeStruct((NPL, E2, WB), jnp.float32),
    )(fa, erd)


def _mm_kernel(a_ref, b_ref, o_ref):
    o_ref[...] = jnp.dot(a_ref[...], b_ref[...], preferred_element_type=jnp.float32)


def _pallas_mm(a, b, bm=2000):
    m, k = a.shape
    n = b.shape[1]
    return pl.pallas_call(
        _mm_kernel,
        grid=(m // bm,),
        in_specs=[
            pl.BlockSpec((bm, k), lambda i: (i, 0)),
            pl.BlockSpec((k, n), lambda i: (0, 0)),
        ],
        out_specs=pl.BlockSpec((bm, n), lambda i: (i, 0)),
        out_shape=jax.ShapeDtypeStruct((m, n), jnp.float32),
    )(a, b)


def _bn(x, g, b):
    m = x.mean(0)
    v = x.var(0)
    return (x - m) / jnp.sqrt(v + 1e-5) * g + b


def _ln(x):
    m = x.mean(-1, keepdims=True)
    v = x.var(-1, keepdims=True)
    return (x - m) / jnp.sqrt(v + 1e-5)


def kernel(x, edge_index, gat_fc_W, gat_fc_b, attn_l, attn_r, sem_W1, sem_b1, sem_W2, gate_W, gate_b, l2W, l2b, l3W, l3b, l4W, l4b, yce_W, yce_b, bn1_g, bn1_b, bn2_g, bn2_b, bn3_g, bn3_b, fus_Wp, fus_b, att_W1, att_b1, att_w2):
    src = edge_index[0]
    dst = edge_index[1]
    ta, tb = _feat_tables(x, gat_fc_W, attn_l, attn_r)
    tb_p = jnp.pad(tb, ((0, NP - N), (0, 0)))
    pad = E2 - E
    src_p = jnp.concatenate([src, jnp.zeros((pad,), jnp.int32)])
    dst_p = jnp.concatenate([dst, jnp.full((pad,), N, jnp.int32)])
    dst3 = dst_p.reshape(_NW, NTS, TS)
    fa, erd = _sc_gather(ta, tb_p, src_p, dst_p)
    mx = _edge_mul(fa, erd)
    zacc = jnp.zeros((STRIPE, WB), jnp.float32)
    parts = _sc_scatter(mx, dst3, zacc).sum(0)
    ssum = parts[NPL - 1][:N, :HEADS]
    agg = parts[:NPL - 1, :N, :]
    rst = agg.reshape(NPL - 1, N, 2, HID).transpose(1, 0, 2, 3).reshape(N, HEADS, HID)
    rst = rst / (ssum[:, :, None] + 1e-9)
    rst = rst + gat_fc_b.reshape(HEADS, HID)[None]
    gat_out = jax.nn.elu(rst).reshape(N, D)
    h3 = _pallas_mm(x, l3W) + l3b
    hh = jax.nn.elu(_bn(h3 + gat_out, bn1_g, bn1_b))
    gate = hh @ gate_W + gate_b
    a = jax.nn.softmax(gate, axis=0)
    Tp = (a * hh).sum(0, keepdims=True)
    T = Tp @ l4W + l4b
    proj = _pallas_mm(hh, fus_Wp) + fus_b
    score = (_ln(x) * _ln(proj)).sum(-1, keepdims=True) / jnp.sqrt(768.0)
    gamma = jax.nn.sigmoid(score)
    newT = _bn(gamma * x + (1.0 - gamma) * proj, bn3_g, bn3_b)
    s0 = jnp.tanh(T @ att_W1 + att_b1) @ att_w2
    s1 = jnp.tanh(_pallas_mm(newT, att_W1) + att_b1) @ att_w2
    sm = jnp.maximum(s0, s1)
    e0 = jnp.exp(s0 - sm)
    e1 = jnp.exp(s1 - sm)
    den = e0 + e1
    RH = (e0 / den)[:, None] * T + (e1 / den)[:, None] * newT
    Tel = jax.nn.elu(_bn(_pallas_mm(RH, l2W) + l2b, bn2_g, bn2_b))
    return Tel @ yce_W + yce_b
